# baseline (speedup 1.0000x reference)
import numpy as np
import jax
import jax.numpy as jnp
from jax.experimental import pallas as pl
from jax.experimental.pallas import tpu as pltpu

N = 32768
D = 512
H = 8
HD = D // H
DF = 4 * D
L = 2
BN = 1024
NB = N // BN
_EPS = 1e-5
_FN = float(N)
_ISQRT2 = np.float32(0.7071067811865476)

_E1_NP = np.kron(np.eye(H, dtype=np.float32), np.ones((HD, 1), np.float32))
_E2_NP = _E1_NP.T.copy()

_BF = jnp.bfloat16


def _qkv_block(h, g1kv, be1kv, g1q, be1q, wq, bq, wk, bk, wv, bv):
    m = jnp.mean(h, axis=1, keepdims=True)
    c = h - m
    var = jnp.mean(c * c, axis=1, keepdims=True)
    cn = (c * jax.lax.rsqrt(var + _EPS)).astype(_BF)
    src = cn * g1kv + be1kv
    qry = cn * g1q + be1q
    q = jnp.dot(qry, wq, preferred_element_type=jnp.float32).astype(_BF) + bq
    k = jnp.dot(src, wk, preferred_element_type=jnp.float32).astype(_BF) + bk
    v = jnp.dot(src, wv, preferred_element_type=jnp.float32).astype(_BF) + bv
    sum_r = lambda a: jnp.sum(a, axis=0, keepdims=True).astype(jnp.float32)
    z = jnp.zeros((1, D), jnp.float32)
    sums = jnp.concatenate([
        sum_r(k * v), sum_r(k), sum_r(q * q), sum_r(k * k),
        z, z, z, z], axis=0)
    return q, v, sums


def _attn_ffn_block(q, v, h, red, e1, e2, wh, bh, g2, be2,
                    wf1, bf1, wf2, bf2):
    q2s = jnp.sum(red[2:3, :])
    k2s = jnp.sum(red[3:4, :])
    rsc = jax.lax.rsqrt(q2s * k2s)
    srow = (red[0:1, :] * rsc).astype(_BF)
    krow = (red[1:2, :] * rsc).astype(_BF)
    num = q * srow + v * _BF(_FN)
    t = q * krow
    den_h = jnp.dot(t, e1, preferred_element_type=jnp.float32)
    den = jnp.dot(den_h.astype(_BF), e2,
                  preferred_element_type=jnp.float32) + _FN
    attn = num / den.astype(_BF)
    hp = jnp.dot(attn, wh, preferred_element_type=jnp.float32) + bh + h
    mm = jnp.mean(hp, axis=1, keepdims=True)
    c2 = hp - mm
    var2 = jnp.mean(c2 * c2, axis=1, keepdims=True)
    zn = (c2 * jax.lax.rsqrt(var2 + _EPS)).astype(_BF) * g2 + be2
    f1 = (jnp.dot(zn, wf1, preferred_element_type=jnp.float32).astype(_BF)
          + bf1)
    f1 = _BF(0.5) * f1 * (_BF(1.0) + jax.lax.erf(f1 * _BF(_ISQRT2)))
    return (jnp.dot(f1, wf2, preferred_element_type=jnp.float32)
            + bf2 + hp)


def _pass_a(h_ref, g1kv_ref, be1kv_ref, g1q_ref, be1q_ref,
            wq_ref, bq_ref, wk_ref, bk_ref, wv_ref, bv_ref,
            q_ref, v_ref, red_ref):
    r = lambda a: a[...]
    q, v, sums = _qkv_block(
        h_ref[...], r(g1kv_ref), r(be1kv_ref), r(g1q_ref), r(be1q_ref),
        r(wq_ref), r(bq_ref), r(wk_ref), r(bk_ref), r(wv_ref), r(bv_ref))
    q_ref[...] = q
    v_ref[...] = v
    red_ref[...] = sums.reshape(1, 8, D)


def _pass_b(q_ref, v_ref, h_ref, rp_ref, e1_ref, e2_ref,
            wh_ref, bh_ref, g2_ref, be2_ref,
            wf1_ref, bf1_ref, wf2_ref, bf2_ref, o_ref):
    r = lambda a: a[...]
    o_ref[...] = _attn_ffn_block(
        q_ref[...], v_ref[...], h_ref[...], rp_ref[0],
        r(e1_ref), r(e2_ref), r(wh_ref), r(bh_ref), r(g2_ref), r(be2_ref),
        r(wf1_ref), r(bf1_ref), r(wf2_ref), r(bf2_ref))


def _pass_ba(q_ref, v_ref, h_ref, rp_ref, e1_ref, e2_ref,
             wh_ref, bh_ref, g2_ref, be2_ref,
             wf1_ref, bf1_ref, wf2_ref, bf2_ref,
             g1kv_ref, be1kv_ref, g1q_ref, be1q_ref,
             wq_ref, bq_ref, wk_ref, bk_ref, wv_ref, bv_ref,
             o_ref, q2_ref, v2_ref, red2_ref):
    r = lambda a: a[...]
    out = _attn_ffn_block(
        q_ref[...], v_ref[...], h_ref[...], rp_ref[0],
        r(e1_ref), r(e2_ref), r(wh_ref), r(bh_ref), r(g2_ref), r(be2_ref),
        r(wf1_ref), r(bf1_ref), r(wf2_ref), r(bf2_ref))
    o_ref[...] = out
    q2, v2, sums2 = _qkv_block(
        out, r(g1kv_ref), r(be1kv_ref), r(g1q_ref), r(be1q_ref),
        r(wq_ref), r(bq_ref), r(wk_ref), r(bk_ref), r(wv_ref), r(bv_ref))
    q2_ref[...] = q2
    v2_ref[...] = v2
    red2_ref[...] = sums2.reshape(1, 8, D)


def _row_spec(w=D):
    return pl.BlockSpec((1, w), lambda n: (0, 0))


def _mat_spec(shape):
    return pl.BlockSpec(shape, lambda n: (0, 0))


def _blk_spec():
    return pl.BlockSpec((BN, D), lambda n: (n, 0))


def _red_spec(moving):
    if moving:
        return pl.BlockSpec((1, 8, D), lambda n: (n, 0, 0))
    return pl.BlockSpec((1, 8, D), lambda n: (0, 0, 0))


def _qkv_specs():
    return [
        _row_spec(), _row_spec(), _row_spec(), _row_spec(),
        _mat_spec((D, D)), _row_spec(),
        _mat_spec((D, D)), _row_spec(),
        _mat_spec((D, D)), _row_spec(),
    ]


def _attn_specs():
    return [
        _blk_spec(), _blk_spec(), _blk_spec(), _red_spec(False),
        _mat_spec((D, H)), _mat_spec((H, D)),
        _mat_spec((D, D)), _row_spec(),
        _row_spec(), _row_spec(),
        _mat_spec((D, DF)), _row_spec(DF),
        _mat_spec((DF, D)), _row_spec(),
    ]


def _qkv_outs():
    return (
        [_blk_spec(), _blk_spec(), _red_spec(True)],
        [jax.ShapeDtypeStruct((N, D), _BF),
         jax.ShapeDtypeStruct((N, D), _BF),
         jax.ShapeDtypeStruct((NB, 8, D), jnp.float32)],
    )


def kernel(x, Wq, bq, Wk, bk, Wv, bv, Wh, bh, g1kv, be1kv, g1q, be1q,
           Wf1, bf1, Wf2, bf2, g2, be2):
    bf = lambda a: a.astype(_BF)
    row = lambda a: a.reshape(1, -1)
    brow = lambda a: bf(a).reshape(1, -1)
    e1 = jnp.asarray(_E1_NP, _BF)
    e2 = jnp.asarray(_E2_NP, _BF)

    def qkv_args(i):
        return (brow(g1kv[i]), brow(be1kv[i]), brow(g1q[i]), brow(be1q[i]),
                bf(Wq[i]), brow(bq[i]), bf(Wk[i]), brow(bk[i]),
                bf(Wv[i]), brow(bv[i]))

    def attn_args(i):
        return (e1, e2, bf(Wh[i]), row(bh[i]), brow(g2[i]), brow(be2[i]),
                bf(Wf1[i]), brow(bf1[i]), bf(Wf2[i]), row(bf2[i]))

    qs, reds = _qkv_outs()

    q1, v1, rp1 = pl.pallas_call(
        _pass_a,
        grid=(NB,),
        in_specs=[_blk_spec()] + _qkv_specs(),
        out_specs=qs,
        out_shape=reds,
    )(x, *qkv_args(0))
    red1 = jnp.sum(rp1, axis=0).reshape(1, 8, D)

    h1, q2, v2, rp2 = pl.pallas_call(
        _pass_ba,
        grid=(NB,),
        in_specs=_attn_specs() + _qkv_specs(),
        out_specs=[pl.BlockSpec((BN, D), lambda n: (n, 0))] + qs,
        out_shape=[jax.ShapeDtypeStruct((N, D), jnp.float32)] + reds,
    )(q1, v1, x, red1, *attn_args(0), *qkv_args(1))
    red2 = jnp.sum(rp2, axis=0).reshape(1, 8, D)

    out = pl.pallas_call(
        _pass_b,
        grid=(NB,),
        in_specs=_attn_specs(),
        out_specs=_blk_spec(),
        out_shape=jax.ShapeDtypeStruct((N, D), jnp.float32),
    )(q2, v2, h1, red2, *attn_args(1))
    return out


# speedup vs baseline: 1.0270x; 1.0270x over previous
import numpy as np
import jax
import jax.numpy as jnp
from jax.experimental import pallas as pl
from jax.experimental.pallas import tpu as pltpu

N = 32768
D = 512
H = 8
HD = D // H
DF = 4 * D
L = 2
BN = 1024
NB = N // BN
_EPS = 1e-5
_FN = float(N)
_ISQRT2 = np.float32(0.7071067811865476)

_E1_NP = np.kron(np.eye(H, dtype=np.float32), np.ones((HD, HD), np.float32))

_BF = jnp.bfloat16


def _qkv_block(h, g1kv, be1kv, g1q, be1q, wq, bq, wk, bk, wv, bv):
    m = jnp.mean(h, axis=1, keepdims=True)
    c = h - m
    var = jnp.mean(c * c, axis=1, keepdims=True)
    cn = (c * jax.lax.rsqrt(var + _EPS)).astype(_BF)
    src = cn * g1kv + be1kv
    qry = cn * g1q + be1q
    q = jnp.dot(qry, wq, preferred_element_type=jnp.float32).astype(_BF) + bq
    k = jnp.dot(src, wk, preferred_element_type=jnp.float32).astype(_BF) + bk
    v = jnp.dot(src, wv, preferred_element_type=jnp.float32).astype(_BF) + bv
    sum_r = lambda a: jnp.sum(a, axis=0, keepdims=True).astype(jnp.float32)
    z = jnp.zeros((1, D), jnp.float32)
    sums = jnp.concatenate([
        sum_r(k * v), sum_r(k), sum_r(q * q), sum_r(k * k),
        z, z, z, z], axis=0)
    return q, v, sums


def _attn_ffn_block(q, v, h, red, e1, wh, bh, g2, be2,
                    wf1, bf1, wf2, bf2):
    q2s = jnp.sum(red[2:3, :])
    k2s = jnp.sum(red[3:4, :])
    rsc = jax.lax.rsqrt(q2s * k2s)
    srow = (red[0:1, :] * rsc).astype(_BF)
    krow = (red[1:2, :] * rsc).astype(_BF)
    num = q * srow + v * _BF(_FN)
    t = q * krow
    den = jnp.dot(t, e1, preferred_element_type=jnp.float32) + _FN
    attn = num / den.astype(_BF)
    hp = jnp.dot(attn, wh, preferred_element_type=jnp.float32) + bh + h
    mm = jnp.mean(hp, axis=1, keepdims=True)
    c2 = hp - mm
    var2 = jnp.mean(c2 * c2, axis=1, keepdims=True)
    zn = (c2 * jax.lax.rsqrt(var2 + _EPS)).astype(_BF) * g2 + be2
    f1 = (jnp.dot(zn, wf1, preferred_element_type=jnp.float32).astype(_BF)
          + bf1)
    f1 = _BF(0.5) * f1 * (_BF(1.0) + jax.lax.erf(f1 * _BF(_ISQRT2)))
    return (jnp.dot(f1, wf2, preferred_element_type=jnp.float32)
            + bf2 + hp)


def _pass_a(h_ref, g1kv_ref, be1kv_ref, g1q_ref, be1q_ref,
            wq_ref, bq_ref, wk_ref, bk_ref, wv_ref, bv_ref,
            q_ref, v_ref, red_ref):
    r = lambda a: a[...]
    q, v, sums = _qkv_block(
        h_ref[...], r(g1kv_ref), r(be1kv_ref), r(g1q_ref), r(be1q_ref),
        r(wq_ref), r(bq_ref), r(wk_ref), r(bk_ref), r(wv_ref), r(bv_ref))
    q_ref[...] = q
    v_ref[...] = v
    red_ref[...] = sums.reshape(1, 8, D)


def _pass_b(q_ref, v_ref, h_ref, rp_ref, e1_ref,
            wh_ref, bh_ref, g2_ref, be2_ref,
            wf1_ref, bf1_ref, wf2_ref, bf2_ref, o_ref):
    r = lambda a: a[...]
    o_ref[...] = _attn_ffn_block(
        q_ref[...], v_ref[...], h_ref[...], rp_ref[0],
        r(e1_ref), r(wh_ref), r(bh_ref), r(g2_ref), r(be2_ref),
        r(wf1_ref), r(bf1_ref), r(wf2_ref), r(bf2_ref))


def _pass_ba(q_ref, v_ref, h_ref, rp_ref, e1_ref,
             wh_ref, bh_ref, g2_ref, be2_ref,
             wf1_ref, bf1_ref, wf2_ref, bf2_ref,
             g1kv_ref, be1kv_ref, g1q_ref, be1q_ref,
             wq_ref, bq_ref, wk_ref, bk_ref, wv_ref, bv_ref,
             o_ref, q2_ref, v2_ref, red2_ref):
    r = lambda a: a[...]
    out = _attn_ffn_block(
        q_ref[...], v_ref[...], h_ref[...], rp_ref[0],
        r(e1_ref), r(wh_ref), r(bh_ref), r(g2_ref), r(be2_ref),
        r(wf1_ref), r(bf1_ref), r(wf2_ref), r(bf2_ref))
    o_ref[...] = out
    q2, v2, sums2 = _qkv_block(
        out, r(g1kv_ref), r(be1kv_ref), r(g1q_ref), r(be1q_ref),
        r(wq_ref), r(bq_ref), r(wk_ref), r(bk_ref), r(wv_ref), r(bv_ref))
    q2_ref[...] = q2
    v2_ref[...] = v2
    red2_ref[...] = sums2.reshape(1, 8, D)


def _row_spec(w=D):
    return pl.BlockSpec((1, w), lambda n: (0, 0))


def _mat_spec(shape):
    return pl.BlockSpec(shape, lambda n: (0, 0))


def _blk_spec():
    return pl.BlockSpec((BN, D), lambda n: (n, 0))


def _red_spec(moving):
    if moving:
        return pl.BlockSpec((1, 8, D), lambda n: (n, 0, 0))
    return pl.BlockSpec((1, 8, D), lambda n: (0, 0, 0))


def _qkv_specs():
    return [
        _row_spec(), _row_spec(), _row_spec(), _row_spec(),
        _mat_spec((D, D)), _row_spec(),
        _mat_spec((D, D)), _row_spec(),
        _mat_spec((D, D)), _row_spec(),
    ]


def _attn_specs():
    return [
        _blk_spec(), _blk_spec(), _blk_spec(), _red_spec(False),
        _mat_spec((D, D)),
        _mat_spec((D, D)), _row_spec(),
        _row_spec(), _row_spec(),
        _mat_spec((D, DF)), _row_spec(DF),
        _mat_spec((DF, D)), _row_spec(),
    ]


def _qkv_outs():
    return (
        [_blk_spec(), _blk_spec(), _red_spec(True)],
        [jax.ShapeDtypeStruct((N, D), _BF),
         jax.ShapeDtypeStruct((N, D), _BF),
         jax.ShapeDtypeStruct((NB, 8, D), jnp.float32)],
    )


def kernel(x, Wq, bq, Wk, bk, Wv, bv, Wh, bh, g1kv, be1kv, g1q, be1q,
           Wf1, bf1, Wf2, bf2, g2, be2):
    bf = lambda a: a.astype(_BF)
    row = lambda a: a.reshape(1, -1)
    brow = lambda a: bf(a).reshape(1, -1)
    e1 = jnp.asarray(_E1_NP, _BF)

    def qkv_args(i):
        return (brow(g1kv[i]), brow(be1kv[i]), brow(g1q[i]), brow(be1q[i]),
                bf(Wq[i]), brow(bq[i]), bf(Wk[i]), brow(bk[i]),
                bf(Wv[i]), brow(bv[i]))

    def attn_args(i):
        return (e1, bf(Wh[i]), row(bh[i]), brow(g2[i]), brow(be2[i]),
                bf(Wf1[i]), brow(bf1[i]), bf(Wf2[i]), row(bf2[i]))

    qs, reds = _qkv_outs()

    q1, v1, rp1 = pl.pallas_call(
        _pass_a,
        grid=(NB,),
        in_specs=[_blk_spec()] + _qkv_specs(),
        out_specs=qs,
        out_shape=reds,
    )(x, *qkv_args(0))
    red1 = jnp.sum(rp1, axis=0).reshape(1, 8, D)

    h1, q2, v2, rp2 = pl.pallas_call(
        _pass_ba,
        grid=(NB,),
        in_specs=_attn_specs() + _qkv_specs(),
        out_specs=[pl.BlockSpec((BN, D), lambda n: (n, 0))] + qs,
        out_shape=[jax.ShapeDtypeStruct((N, D), jnp.float32)] + reds,
    )(q1, v1, x, red1, *attn_args(0), *qkv_args(1))
    red2 = jnp.sum(rp2, axis=0).reshape(1, 8, D)

    out = pl.pallas_call(
        _pass_b,
        grid=(NB,),
        in_specs=_attn_specs(),
        out_specs=_blk_spec(),
        out_shape=jax.ShapeDtypeStruct((N, D), jnp.float32),
    )(q2, v2, h1, red2, *attn_args(1))
    return out
